# baseline (speedup 1.0000x reference)
import functools

import jax
import jax.numpy as jnp
from jax import lax
from jax.experimental import pallas as pl
from jax.experimental.pallas import tpu as pltpu
from jax.experimental.pallas import tpu_sc as plsc

D_MODEL = 64
SEQ_LEN = 200
BATCH = 1024
VOC = 1000000
NTOK = BATCH * SEQ_LEN

_info = plsc.get_sparse_core_info()
NC, NS = _info.num_cores, _info.num_subcores
NW = NC * NS
VLANES = 16

VBLK = 128
NBLK = VOC // VBLK
BPW = (NBLK + NW - 1) // NW
NSLOT = BPW + (BPW % 2)
TAIL0 = NBLK * VBLK
TAILN = VOC - TAIL0


@functools.partial(
    pl.kernel,
    out_type=jax.ShapeDtypeStruct((VOC // 2, 128), jnp.float32),
    mesh=plsc.VectorSubcoreMesh(core_axis_name="c", subcore_axis_name="s"),
    scratch_types=[
        pltpu.VMEM((2, D_MODEL, VBLK), jnp.float32),
        pltpu.VMEM((2, VBLK // 2, 128), jnp.float32),
        pltpu.SemaphoreType.DMA,
        pltpu.SemaphoreType.DMA,
        pltpu.SemaphoreType.DMA,
        pltpu.SemaphoreType.DMA,
    ],
    compiler_params=pltpu.CompilerParams(needs_layout_passes=False),
)
def _compact_table(tabt_hbm, tail_hbm, out_hbm, in_v, op_v,
                   sem_in0, sem_in1, sem_out0, sem_out1):
    sem_ins = (sem_in0, sem_in1)
    sem_outs = (sem_out0, sem_out1)
    wid = lax.axis_index("s") * NC + lax.axis_index("c")

    def blk_id(slot):
        return jnp.minimum(wid + slot * NW, NBLK - 1)

    def fetch(slot, b):
        pltpu.async_copy(
            tabt_hbm.at[:, pl.ds(blk_id(slot) * VBLK, VBLK)],
            in_v.at[b], sem_ins[b])

    def transpose_block(b):
        row_idx = [jnp.arange(c4 * VLANES, (c4 + 1) * VLANES, dtype=jnp.int32)
                   for c4 in range(D_MODEL // VLANES)]
        ib, ob = in_v.at[b], op_v.at[b]

        def body(p, carry):
            for q in range(2):
                col = jnp.broadcast_to(2 * p + q, (VLANES,)).astype(jnp.int32)
                for c4 in range(D_MODEL // VLANES):
                    vec = plsc.load_gather(ib, [row_idx[c4], col])
                    ob[p, pl.ds(q * D_MODEL + c4 * VLANES, VLANES)] = vec
            return carry

        lax.fori_loop(0, VBLK // 2, body, 0)

    def flush(slot, b):
        pltpu.async_copy(
            op_v.at[b],
            out_hbm.at[pl.ds(blk_id(slot) * (VBLK // 2), VBLK // 2)],
            sem_outs[b])

    def wait_in(b):
        pltpu.make_async_copy(tabt_hbm.at[:, pl.ds(0, VBLK)], in_v.at[b],
                              sem_ins[b]).wait()

    def wait_out(b):
        pltpu.make_async_copy(op_v.at[b], out_hbm.at[pl.ds(0, VBLK // 2)],
                              sem_outs[b]).wait()

    fetch(0, 0)
    fetch(1, 1)
    for b in range(2):
        wait_in(b)
        transpose_block(b)
        flush(b, b)
        fetch(2 + b, b)

    def pair(jj, carry):
        for b in range(2):
            slot = 2 * jj + b
            wait_in(b)
            wait_out(b)
            transpose_block(b)
            flush(slot, b)
            fetch(slot + 2, b)
        return carry

    lax.fori_loop(1, NSLOT // 2 - 1, pair, 0)

    for b in range(2):
        slot = NSLOT - 2 + b
        wait_in(b)
        wait_out(b)
        transpose_block(b)
        flush(slot, b)
    for b in range(2):
        wait_out(b)

    pltpu.sync_copy(tail_hbm, in_v.at[0])
    row_idx = [jnp.arange(c4 * VLANES, (c4 + 1) * VLANES, dtype=jnp.int32)
               for c4 in range(D_MODEL // VLANES)]

    def tail_body(p, carry):
        for q in range(2):
            col = jnp.broadcast_to(2 * p + q, (VLANES,)).astype(jnp.int32)
            for c4 in range(D_MODEL // VLANES):
                vec = plsc.load_gather(in_v.at[0], [row_idx[c4], col])
                op_v.at[0][p, pl.ds(q * D_MODEL + c4 * VLANES, VLANES)] = vec
        return carry

    lax.fori_loop(0, TAILN // 2, tail_body, 0)
    pltpu.sync_copy(op_v.at[0].at[pl.ds(0, TAILN // 2)],
                    out_hbm.at[pl.ds(TAIL0 // 2, TAILN // 2)])


CHUNK = SEQ_LEN
NCHUNK = NTOK // NW // CHUNK
SUB = 40
NSUB = CHUNK // SUB
NCOL = D_MODEL // VLANES


@functools.partial(
    pl.kernel,
    out_type=jax.ShapeDtypeStruct((BATCH, SEQ_LEN, D_MODEL), jnp.float32),
    mesh=plsc.VectorSubcoreMesh(core_axis_name="c", subcore_axis_name="s"),
    scratch_types=[
        pltpu.VMEM((SEQ_LEN, D_MODEL), jnp.float32),
        pltpu.VMEM((2, NSUB, SUB), jnp.int32),
        pltpu.VMEM((2, CHUNK, D_MODEL), jnp.float32),
        pltpu.SemaphoreType.DMA,
        pltpu.SemaphoreType.DMA,
    ],
    compiler_params=pltpu.CompilerParams(use_tc_tiling_on_sc=False),
)
def _emb_lookup(idx_hbm, table_hbm, pos_hbm, out_hbm, pos_v, idx_v, rows_v,
                sem_a, sem_b):
    sems = (sem_a, sem_b)
    wid = lax.axis_index("s") * NC + lax.axis_index("c")

    pltpu.sync_copy(pos_hbm, pos_v)

    def fire(k, b):
        base = wid * (NCHUNK * CHUNK) + k * CHUNK
        for j in range(NSUB):
            pltpu.sync_copy(idx_hbm.at[pl.ds(base + j * SUB, SUB)],
                            idx_v.at[b].at[j])
        for j in range(NSUB):
            pltpu.async_copy(
                table_hbm.at[idx_v.at[b].at[j]],
                rows_v.at[b].at[pl.ds(j * SUB, SUB)],
                sems[b],
            )

    def drain(b):
        pltpu.make_async_copy(
            table_hbm.at[pl.ds(0, CHUNK)], rows_v.at[b], sems[b]
        ).wait()

    def add_pos(b):
        rb = rows_v.at[b]

        def body(r, carry):
            for c in range(NCOL):
                sl = pl.ds(c * VLANES, VLANES)
                rb[r, sl] += pos_v[r, sl]
            return carry

        lax.fori_loop(0, CHUNK, body, 0)

    def store(k, b):
        seq = wid * NCHUNK + k
        pltpu.sync_copy(rows_v.at[b], out_hbm.at[seq])

    fire(0, 0)
    for k in range(NCHUNK):
        b = k & 1
        if k + 1 < NCHUNK:
            fire(k + 1, 1 - b)
        drain(b)
        add_pos(b)
        store(k, b)


def kernel(inputs, input_emb_table, positional_emb_table):
    tabt = jnp.transpose(input_emb_table)
    tail = jnp.pad(tabt[:, TAIL0:], ((0, 0), (0, VBLK - TAILN)))
    packed = _compact_table(tabt, tail)
    table_lin = packed.reshape(VOC, D_MODEL)
    idx = inputs.astype(jnp.int32).reshape(NTOK)
    return _emb_lookup(idx, table_lin, positional_emb_table)


# speedup vs baseline: 1.5582x; 1.5582x over previous
import functools

import jax
import jax.numpy as jnp
from jax import lax
from jax.experimental import pallas as pl
from jax.experimental.pallas import tpu as pltpu
from jax.experimental.pallas import tpu_sc as plsc

D_MODEL = 64
SEQ_LEN = 200
BATCH = 1024
VOC = 1000000
NTOK = BATCH * SEQ_LEN

_info = plsc.get_sparse_core_info()
NC, NS = _info.num_cores, _info.num_subcores
NW = NC * NS
VLANES = 16

VBLK = 128
NBLK = VOC // VBLK
BPW = (NBLK + NW - 1) // NW
NSLOT = BPW + (BPW % 2)
TAIL0 = NBLK * VBLK
TAILN = VOC - TAIL0


@functools.partial(
    pl.kernel,
    out_type=jax.ShapeDtypeStruct((VOC // 2, 128), jnp.float32),
    mesh=plsc.VectorSubcoreMesh(core_axis_name="c", subcore_axis_name="s"),
    scratch_types=[
        pltpu.VMEM((2, D_MODEL, VBLK), jnp.float32),
        pltpu.VMEM((2, VBLK // 2, 128), jnp.float32),
        pltpu.SemaphoreType.DMA,
        pltpu.SemaphoreType.DMA,
        pltpu.SemaphoreType.DMA,
        pltpu.SemaphoreType.DMA,
    ],
    compiler_params=pltpu.CompilerParams(needs_layout_passes=False),
)
def _compact_table(tabt_hbm, tail_hbm, out_hbm, in_v, op_v,
                   sem_in0, sem_in1, sem_out0, sem_out1):
    sem_ins = (sem_in0, sem_in1)
    sem_outs = (sem_out0, sem_out1)
    wid = lax.axis_index("s") * NC + lax.axis_index("c")

    def blk_id(slot):
        return jnp.minimum(wid + slot * NW, NBLK - 1)

    def fetch(slot, b):
        pltpu.async_copy(
            tabt_hbm.at[:, pl.ds(blk_id(slot) * VBLK, VBLK)],
            in_v.at[b], sem_ins[b])

    _v = [jnp.arange(c * VLANES, (c + 1) * VLANES, dtype=jnp.int32)
          for c in range(VBLK // VLANES)]
    row_c = [v >> 1 for v in _v]
    colp_c = [(v & 1) * D_MODEL for v in _v]

    def transpose_block(b):
        ib, ob = in_v.at[b], op_v.at[b]

        @plsc.parallel_loop(0, D_MODEL, unroll=2)
        def _(f):
            fv = jnp.broadcast_to(f, (VLANES,)).astype(jnp.int32)
            for c in range(VBLK // VLANES):
                vec = ib[f, pl.ds(c * VLANES, VLANES)]
                plsc.store_scatter(ob, [row_c[c], colp_c[c] + fv], vec)

    def flush(slot, b):
        pltpu.async_copy(
            op_v.at[b],
            out_hbm.at[pl.ds(blk_id(slot) * (VBLK // 2), VBLK // 2)],
            sem_outs[b])

    def wait_in(b):
        pltpu.make_async_copy(tabt_hbm.at[:, pl.ds(0, VBLK)], in_v.at[b],
                              sem_ins[b]).wait()

    def wait_out(b):
        pltpu.make_async_copy(op_v.at[b], out_hbm.at[pl.ds(0, VBLK // 2)],
                              sem_outs[b]).wait()

    fetch(0, 0)
    fetch(1, 1)
    for b in range(2):
        wait_in(b)
        transpose_block(b)
        flush(b, b)
        fetch(2 + b, b)

    def pair(jj, carry):
        for b in range(2):
            slot = 2 * jj + b
            wait_in(b)
            wait_out(b)
            transpose_block(b)
            flush(slot, b)
            fetch(slot + 2, b)
        return carry

    lax.fori_loop(1, NSLOT // 2 - 1, pair, 0)

    for b in range(2):
        slot = NSLOT - 2 + b
        wait_in(b)
        wait_out(b)
        transpose_block(b)
        flush(slot, b)
    for b in range(2):
        wait_out(b)

    pltpu.sync_copy(tail_hbm, in_v.at[0])
    row_idx = [jnp.arange(c4 * VLANES, (c4 + 1) * VLANES, dtype=jnp.int32)
               for c4 in range(D_MODEL // VLANES)]

    def tail_body(p, carry):
        for q in range(2):
            col = jnp.broadcast_to(2 * p + q, (VLANES,)).astype(jnp.int32)
            for c4 in range(D_MODEL // VLANES):
                vec = plsc.load_gather(in_v.at[0], [row_idx[c4], col])
                op_v.at[0][p, pl.ds(q * D_MODEL + c4 * VLANES, VLANES)] = vec
        return carry

    lax.fori_loop(0, TAILN // 2, tail_body, 0)
    pltpu.sync_copy(op_v.at[0].at[pl.ds(0, TAILN // 2)],
                    out_hbm.at[pl.ds(TAIL0 // 2, TAILN // 2)])


CHUNK = SEQ_LEN
NCHUNK = NTOK // NW // CHUNK
SUB = 40
NSUB = CHUNK // SUB
NCOL = D_MODEL // VLANES


@functools.partial(
    pl.kernel,
    out_type=jax.ShapeDtypeStruct((BATCH, SEQ_LEN, D_MODEL), jnp.float32),
    mesh=plsc.VectorSubcoreMesh(core_axis_name="c", subcore_axis_name="s"),
    scratch_types=[
        pltpu.VMEM((SEQ_LEN, D_MODEL), jnp.float32),
        pltpu.VMEM((2, NSUB, SUB), jnp.int32),
        pltpu.VMEM((2, CHUNK, D_MODEL), jnp.float32),
        pltpu.SemaphoreType.DMA,
        pltpu.SemaphoreType.DMA,
    ],
    compiler_params=pltpu.CompilerParams(use_tc_tiling_on_sc=False),
)
def _emb_lookup(idx_hbm, table_hbm, pos_hbm, out_hbm, pos_v, idx_v, rows_v,
                sem_a, sem_b):
    sems = (sem_a, sem_b)
    wid = lax.axis_index("s") * NC + lax.axis_index("c")

    pltpu.sync_copy(pos_hbm, pos_v)

    def fire(k, b):
        base = wid * (NCHUNK * CHUNK) + k * CHUNK
        for j in range(NSUB):
            pltpu.sync_copy(idx_hbm.at[pl.ds(base + j * SUB, SUB)],
                            idx_v.at[b].at[j])
        for j in range(NSUB):
            pltpu.async_copy(
                table_hbm.at[idx_v.at[b].at[j]],
                rows_v.at[b].at[pl.ds(j * SUB, SUB)],
                sems[b],
            )

    def drain(b):
        pltpu.make_async_copy(
            table_hbm.at[pl.ds(0, CHUNK)], rows_v.at[b], sems[b]
        ).wait()

    def add_pos(b):
        rb = rows_v.at[b]

        def body(r, carry):
            for c in range(NCOL):
                sl = pl.ds(c * VLANES, VLANES)
                rb[r, sl] += pos_v[r, sl]
            return carry

        lax.fori_loop(0, CHUNK, body, 0)

    def store(k, b):
        seq = wid * NCHUNK + k
        pltpu.sync_copy(rows_v.at[b], out_hbm.at[seq])

    fire(0, 0)
    for k in range(NCHUNK):
        b = k & 1
        if k + 1 < NCHUNK:
            fire(k + 1, 1 - b)
        drain(b)
        add_pos(b)
        store(k, b)


def kernel(inputs, input_emb_table, positional_emb_table):
    tabt = jnp.transpose(input_emb_table)
    tail = jnp.pad(tabt[:, TAIL0:], ((0, 0), (0, VBLK - TAILN)))
    packed = _compact_table(tabt, tail)
    table_lin = packed.reshape(VOC, D_MODEL)
    idx = inputs.astype(jnp.int32).reshape(NTOK)
    return _emb_lookup(idx, table_lin, positional_emb_table)
